# baseline (speedup 1.0000x reference)
import dataclasses
import functools

import jax
import jax.numpy as jnp
import numpy as np
from jax import lax
from jax.experimental import pallas as pl
from jax.experimental.pallas import tpu as pltpu
from jax.experimental.pallas import tpu_sc as plsc

_KERNEL_SIZE = 5
_PADDING = 4

_NC = 2
_NS = 16
_LANES = 16

_SC_ROWS = 1024
_TC_BLK = 128


def _compiler_params():
    cp = pltpu.CompilerParams(use_tc_tiling_on_sc=True)
    if "needs_layout_passes" in pltpu.CompilerParams.__dataclass_fields__:
        cp = dataclasses.replace(cp, needs_layout_passes=False)
    return cp


@functools.lru_cache(maxsize=None)
def _tables(B, W, S, WP):
    with jax.ensure_compile_time_eval():
        idx = np.asarray(
            jax.random.randint(
                jax.random.key(1), (B, W), 0, _KERNEL_SIZE, dtype=jnp.int32
            )
        )
    w_ar = np.arange(W, dtype=np.int32)
    src = idx * W + w_ar[None, :] - _PADDING
    valid = (src >= 0) & (src < S)
    src_pad = np.zeros((B, WP), np.int32)
    src_pad[:, :W] = np.where(valid, src, 0)
    mask = np.zeros((B, WP), np.float32)
    mask[:, :W] = valid.astype(np.float32)
    sel = idx[:, None, :] == np.arange(_KERNEL_SIZE)[None, :, None]
    selm = np.zeros((B, _KERNEL_SIZE, WP), np.float32)
    selm[:, :, :W] = (sel & valid[:, None, :]).astype(np.float32)
    return src_pad, mask, selm


def kernel(x):
    B, F, S = x.shape
    W = (S + 2 * _PADDING) // _KERNEL_SIZE
    WP = ((W + 127) // 128) * 128
    CT = WP // 128

    src_pad, mask, selm = _tables(B, W, S, WP)
    src_pad, mask, selm = (
        jnp.asarray(src_pad), jnp.asarray(mask), jnp.asarray(selm),
    )

    NW = _NC * _NS
    R = B * F
    NBLK = R // 8
    TC_ROWS = R - _SC_ROWS
    SC_BLKS = _SC_ROWS // 8
    SC_OFF_BLK = TC_ROWS // 8
    blocks_per_tile = SC_BLKS // NW
    x3 = x.reshape(NBLK, 8, S)
    x2 = x.reshape(R, S)

    mesh = plsc.VectorSubcoreMesh(core_axis_name="c", subcore_axis_name="s")

    @functools.partial(
        pl.kernel,
        out_type=jax.ShapeDtypeStruct((_SC_ROWS * CT, 128), x.dtype),
        mesh=mesh,
        compiler_params=_compiler_params(),
        scratch_types=[
            pltpu.VMEM((B, WP), jnp.int32),
            pltpu.VMEM((B, WP), jnp.float32),
            pltpu.VMEM((8, S), jnp.float32),
            pltpu.VMEM((8 * CT, 128), jnp.float32),
            pltpu.VMEM((8 * CT, 128), jnp.float32),
            pltpu.SemaphoreType.DMA,
            pltpu.SemaphoreType.DMA,
        ],
    )
    def sc_gather_blocks(x_hbm, src_hbm, msk_hbm, o_hbm, src_v, msk_v, ib,
                         ob0, ob1, osem0, osem1):
        wid = lax.axis_index("s") * _NC + lax.axis_index("c")
        base = SC_OFF_BLK + wid * blocks_per_tile
        bt = (base * 8) // F
        pltpu.sync_copy(src_hbm, src_v)
        pltpu.sync_copy(msk_hbm, msk_v)

        def out_copy(t, obuf, sem):
            return pltpu.make_async_copy(
                obuf,
                o_hbm.at[pl.ds((base - SC_OFF_BLK + t) * 8 * CT, 8 * CT), :],
                sem,
            )

        def do_block(t, obuf):
            pltpu.sync_copy(x_hbm.at[base + t], ib)

            @pl.loop(0, CT)
            def _(c):
                for u in range(8):
                    iv = src_v[bt, pl.ds(c * 128 + u * 16, _LANES)]
                    mv = msk_v[bt, pl.ds(c * 128 + u * 16, _LANES)]
                    for fs in range(8):
                        rowv = jnp.full((_LANES,), fs, jnp.int32)
                        g = plsc.load_gather(ib, [rowv, iv])
                        obuf[fs * CT + c, pl.ds(u * 16, _LANES)] = g * mv

        @pl.loop(0, blocks_per_tile, step=2)
        def _(t):
            @pl.when(t >= 2)
            def _():
                out_copy(t - 2, ob0, osem0).wait()

            do_block(t, ob0)
            out_copy(t, ob0, osem0).start()

            @pl.when(t >= 2)
            def _():
                out_copy(t - 1, ob1, osem1).wait()

            do_block(t + 1, ob1)
            out_copy(t + 1, ob1, osem1).start()

        out_copy(blocks_per_tile - 2, ob0, osem0).wait()
        out_copy(blocks_per_tile - 1, ob1, osem1).wait()

    FB = F // _TC_BLK

    def tc_select_body(x_ref, m_ref, o_ref):
        xb = x_ref[...]
        z4 = jnp.zeros((_TC_BLK, _PADDING), jnp.float32)
        acc = None
        for k in range(_KERNEL_SIZE):
            lo = k * W - _PADDING
            if lo < 0:
                cand = jnp.concatenate([z4, xb[:, : W + lo]], axis=1)
            elif lo + W > S:
                cand = jnp.concatenate([xb[:, lo:S], z4], axis=1)
            else:
                cand = xb[:, lo : lo + W]
            term = cand * m_ref[0, k : k + 1, :W]
            acc = term if acc is None else acc + term
        o_ref[0] = acc.T

    y_tc = pl.pallas_call(
        tc_select_body,
        grid=(TC_ROWS // _TC_BLK,),
        in_specs=[
            pl.BlockSpec((_TC_BLK, S), lambda i: (i, 0)),
            pl.BlockSpec(
                (1, _KERNEL_SIZE, WP), lambda i: (i // FB, 0, 0)
            ),
        ],
        out_specs=pl.BlockSpec(
            (1, W, _TC_BLK), lambda i: (i // FB, 0, i % FB)
        ),
        out_shape=jax.ShapeDtypeStruct((B, W, F), x.dtype),
    )(x2, selm)

    y_sc = sc_gather_blocks(x3, src_pad, mask)

    SC_FB0 = (TC_ROWS - (B - 1) * F) // _TC_BLK

    def fold_body(ysc_ref, yfull_ref, o_ref):
        del yfull_ref
        yb = ysc_ref[...].reshape(_TC_BLK, WP)
        o_ref[0] = yb[:, :W].T

    out = pl.pallas_call(
        fold_body,
        grid=(_SC_ROWS // _TC_BLK,),
        in_specs=[
            pl.BlockSpec((_TC_BLK * CT, 128), lambda i: (i, 0)),
            pl.BlockSpec((1, 8, 128), lambda i: (0, 0, 0)),
        ],
        out_specs=pl.BlockSpec(
            (1, W, _TC_BLK), lambda i: (B - 1, 0, i + SC_FB0)
        ),
        out_shape=jax.ShapeDtypeStruct((B, W, F), x.dtype),
        input_output_aliases={1: 0},
    )(y_sc, y_tc)

    return out.transpose(0, 2, 1)


# speedup vs baseline: 1.0927x; 1.0927x over previous
import dataclasses
import functools

import jax
import jax.numpy as jnp
import numpy as np
from jax import lax
from jax.experimental import pallas as pl
from jax.experimental.pallas import tpu as pltpu
from jax.experimental.pallas import tpu_sc as plsc

_KERNEL_SIZE = 5
_PADDING = 4

_NC = 2
_NS = 16
_LANES = 16

_SC_ROWS = 1024
_TC_BLK = 256


def _compiler_params():
    cp = pltpu.CompilerParams(use_tc_tiling_on_sc=True)
    if "needs_layout_passes" in pltpu.CompilerParams.__dataclass_fields__:
        cp = dataclasses.replace(cp, needs_layout_passes=False)
    return cp


@functools.lru_cache(maxsize=None)
def _tables(B, W, S, WP):
    with jax.ensure_compile_time_eval():
        idx = np.asarray(
            jax.random.randint(
                jax.random.key(1), (B, W), 0, _KERNEL_SIZE, dtype=jnp.int32
            )
        )
    w_ar = np.arange(W, dtype=np.int32)
    src = idx * W + w_ar[None, :] - _PADDING
    valid = (src >= 0) & (src < S)
    src_pad = np.zeros((B, WP), np.int32)
    src_pad[:, :W] = np.where(valid, src, 0)
    mask = np.zeros((B, WP), np.float32)
    mask[:, :W] = valid.astype(np.float32)
    sel = idx[:, None, :] == np.arange(_KERNEL_SIZE)[None, :, None]
    selm = np.zeros((B, _KERNEL_SIZE, WP), np.float32)
    selm[:, :, :W] = (sel & valid[:, None, :]).astype(np.float32)
    return src_pad, mask, selm


def kernel(x):
    B, F, S = x.shape
    W = (S + 2 * _PADDING) // _KERNEL_SIZE
    WP = ((W + 127) // 128) * 128
    CT = WP // 128

    src_pad, mask, selm = _tables(B, W, S, WP)
    src_pad, mask, selm = (
        jnp.asarray(src_pad), jnp.asarray(mask), jnp.asarray(selm),
    )

    NW = _NC * _NS
    R = B * F
    NBLK = R // 8
    TC_ROWS = R - _SC_ROWS
    SC_BLKS = _SC_ROWS // 8
    SC_OFF_BLK = TC_ROWS // 8
    blocks_per_tile = SC_BLKS // NW
    x3 = x.reshape(NBLK, 8, S)
    x2 = x.reshape(R, S)

    mesh = plsc.VectorSubcoreMesh(core_axis_name="c", subcore_axis_name="s")

    @functools.partial(
        pl.kernel,
        out_type=jax.ShapeDtypeStruct((_SC_ROWS * CT, 128), x.dtype),
        mesh=mesh,
        compiler_params=_compiler_params(),
        scratch_types=[
            pltpu.VMEM((B, WP), jnp.int32),
            pltpu.VMEM((B, WP), jnp.float32),
            pltpu.VMEM((8, S), jnp.float32),
            pltpu.VMEM((8 * CT, 128), jnp.float32),
            pltpu.VMEM((8 * CT, 128), jnp.float32),
            pltpu.SemaphoreType.DMA,
            pltpu.SemaphoreType.DMA,
        ],
    )
    def sc_gather_blocks(x_hbm, src_hbm, msk_hbm, o_hbm, src_v, msk_v, ib,
                         ob0, ob1, osem0, osem1):
        wid = lax.axis_index("s") * _NC + lax.axis_index("c")
        base = SC_OFF_BLK + wid * blocks_per_tile
        bt = (base * 8) // F
        pltpu.sync_copy(src_hbm, src_v)
        pltpu.sync_copy(msk_hbm, msk_v)

        def out_copy(t, obuf, sem):
            return pltpu.make_async_copy(
                obuf,
                o_hbm.at[pl.ds((base - SC_OFF_BLK + t) * 8 * CT, 8 * CT), :],
                sem,
            )

        def do_block(t, obuf):
            pltpu.sync_copy(x_hbm.at[base + t], ib)

            @pl.loop(0, CT)
            def _(c):
                for u in range(8):
                    iv = src_v[bt, pl.ds(c * 128 + u * 16, _LANES)]
                    mv = msk_v[bt, pl.ds(c * 128 + u * 16, _LANES)]
                    for fs in range(8):
                        rowv = jnp.full((_LANES,), fs, jnp.int32)
                        g = plsc.load_gather(ib, [rowv, iv])
                        obuf[fs * CT + c, pl.ds(u * 16, _LANES)] = g * mv

        @pl.loop(0, blocks_per_tile, step=2)
        def _(t):
            @pl.when(t >= 2)
            def _():
                out_copy(t - 2, ob0, osem0).wait()

            do_block(t, ob0)
            out_copy(t, ob0, osem0).start()

            @pl.when(t >= 2)
            def _():
                out_copy(t - 1, ob1, osem1).wait()

            do_block(t + 1, ob1)
            out_copy(t + 1, ob1, osem1).start()

        out_copy(blocks_per_tile - 2, ob0, osem0).wait()
        out_copy(blocks_per_tile - 1, ob1, osem1).wait()

    FB = F // _TC_BLK

    def tc_select_body(x_ref, m_ref, o_ref):
        xb = x_ref[...]
        z4 = jnp.zeros((_TC_BLK, _PADDING), jnp.float32)
        acc = None
        for k in range(_KERNEL_SIZE):
            lo = k * W - _PADDING
            if lo < 0:
                cand = jnp.concatenate([z4, xb[:, : W + lo]], axis=1)
            elif lo + W > S:
                cand = jnp.concatenate([xb[:, lo:S], z4], axis=1)
            else:
                cand = xb[:, lo : lo + W]
            term = cand * m_ref[0, k : k + 1, :W]
            acc = term if acc is None else acc + term
        o_ref[0] = acc.T

    y_tc = pl.pallas_call(
        tc_select_body,
        grid=(TC_ROWS // _TC_BLK,),
        in_specs=[
            pl.BlockSpec((_TC_BLK, S), lambda i: (i, 0)),
            pl.BlockSpec(
                (1, _KERNEL_SIZE, WP), lambda i: (i // FB, 0, 0)
            ),
        ],
        out_specs=pl.BlockSpec(
            (1, W, _TC_BLK), lambda i: (i // FB, 0, i % FB)
        ),
        out_shape=jax.ShapeDtypeStruct((B, W, F), x.dtype),
    )(x2, selm)

    y_sc = sc_gather_blocks(x3, src_pad, mask)

    SC_FB0 = (TC_ROWS - (B - 1) * F) // _TC_BLK

    def fold_body(ysc_ref, yfull_ref, o_ref):
        del yfull_ref
        yb = ysc_ref[...].reshape(_TC_BLK, WP)
        o_ref[0] = yb[:, :W].T

    out = pl.pallas_call(
        fold_body,
        grid=(_SC_ROWS // _TC_BLK,),
        in_specs=[
            pl.BlockSpec((_TC_BLK * CT, 128), lambda i: (i, 0)),
            pl.BlockSpec((1, 8, 128), lambda i: (0, 0, 0)),
        ],
        out_specs=pl.BlockSpec(
            (1, W, _TC_BLK), lambda i: (B - 1, 0, i + SC_FB0)
        ),
        out_shape=jax.ShapeDtypeStruct((B, W, F), x.dtype),
        input_output_aliases={1: 0},
    )(y_sc, y_tc)

    return out.transpose(0, 2, 1)
